# baseline (speedup 1.0000x reference)
import functools

import jax
import jax.numpy as jnp
from jax import lax
from jax.experimental import pallas as pl
from jax.experimental.pallas import tpu as pltpu
from jax.experimental.pallas import tpu_sc as plsc

VOCAB = 1_000_000
HIDDEN = 64
BATCH = 4096
HIST = 200

NUM_CORES = 2
NUM_SUBCORES = 16
NW = NUM_CORES * NUM_SUBCORES
BBLK = BATCH // NW
HB = HIDDEN // 8

_mesh = plsc.VectorSubcoreMesh(core_axis_name="c", subcore_axis_name="s")


@functools.partial(
    pl.kernel,
    mesh=_mesh,
    compiler_params=pltpu.CompilerParams(
        use_tc_tiling_on_sc=False, needs_layout_passes=False
    ),
    out_type=jax.ShapeDtypeStruct((HIST, HB, NW, 8, 128), jnp.float32),
    scratch_types=[
        pltpu.VMEM((HIST, BBLK), jnp.int32),
        pltpu.VMEM((BBLK, HIDDEN), jnp.float32),
        pltpu.VMEM((BBLK, HIDDEN), jnp.float32),
        pltpu.VMEM((HB, 8, 128), jnp.float32),
        pltpu.VMEM((HB, 8, 128), jnp.float32),
        pltpu.SemaphoreType.DMA((2,)),
        pltpu.SemaphoreType.DMA((2,)),
    ],
)
def _emb_lookup(
    ids_hbm, table_hbm, out_hbm, idx_v, gbuf0, gbuf1, tbuf0, tbuf1, gsem, osem
):
    gbufs = (gbuf0, gbuf1)
    tbufs = (tbuf0, tbuf1)
    wid = lax.axis_index("s") * NUM_CORES + lax.axis_index("c")
    pltpu.sync_copy(ids_hbm.at[:, pl.ds(wid * BBLK, BBLK)], idx_v)

    iota16 = lax.iota(jnp.int32, 16)

    def g_copy(t, s):
        return pltpu.make_async_copy(
            table_hbm.at[idx_v.at[t]], gbufs[s], gsem.at[s]
        )

    def o_copies(t, s):
        return [
            pltpu.make_async_copy(
                tbufs[s].at[hb], out_hbm.at[t, hb, wid], osem.at[s]
            )
            for hb in range(HB)
        ]

    def transpose(s):
        gbuf = gbufs[s]
        tbuf = tbufs[s]

        def jb_body(jb, _):
            rows = jb * 16 + iota16
            start = jb * 16
            for h in range(HIDDEN):
                v = plsc.load_gather(
                    gbuf, [rows, jnp.full((16,), h, jnp.int32)]
                )
                tbuf[h // 8, h % 8, pl.ds(start, 16)] = v
            return ()

        lax.fori_loop(0, BBLK // 16, jb_body, (), unroll=False)

    def step(t, s, *, first=False, last=False):
        g_copy(t, s).wait()
        if not last:
            g_copy(t + 1, 1 - s).start()
        if not first:
            for c in o_copies(t - 2, s):
                c.wait()
        transpose(s)
        for c in o_copies(t, s):
            c.start()

    g_copy(0, 0).start()
    step(0, 0, first=True)
    step(1, 1, first=True)

    def steady(t2, _):
        t = t2 * 2
        step(t, 0)
        step(t + 1, 1)
        return ()

    lax.fori_loop(1, (HIST - 2) // 2, steady, (), unroll=False)

    step(HIST - 2, 0)
    step(HIST - 1, 1, last=True)
    for s in (0, 1):
        for c in o_copies(HIST - 2 + s, s):
            c.wait()


def kernel(weight, input_ids):
    ids_t = input_ids.T.astype(jnp.int32)
    z = _emb_lookup(ids_t, weight)
    return z.transpose(2, 4, 0, 1, 3).reshape(BATCH, HIST, HIDDEN)


# speedup vs baseline: 1.1406x; 1.1406x over previous
import functools

import jax
import jax.numpy as jnp
from jax import lax
from jax.experimental import pallas as pl
from jax.experimental.pallas import tpu as pltpu
from jax.experimental.pallas import tpu_sc as plsc

VOCAB = 1_000_000
HIDDEN = 64
BATCH = 4096
HIST = 200

NUM_CORES = 2
NUM_SUBCORES = 16
NW = NUM_CORES * NUM_SUBCORES
BBLK = BATCH // NW
HB = HIDDEN // 8

_mesh = plsc.VectorSubcoreMesh(core_axis_name="c", subcore_axis_name="s")


@functools.partial(
    pl.kernel,
    mesh=_mesh,
    compiler_params=pltpu.CompilerParams(
        use_tc_tiling_on_sc=False, needs_layout_passes=False
    ),
    out_type=jax.ShapeDtypeStruct((HIST, HB, NW, 8, 128), jnp.float32),
    scratch_types=[
        pltpu.VMEM((HIST, BBLK), jnp.int32),
        pltpu.VMEM((BBLK, HIDDEN), jnp.float32),
        pltpu.VMEM((BBLK, HIDDEN), jnp.float32),
        pltpu.VMEM((HB, 8, 128), jnp.float32),
        pltpu.VMEM((HB, 8, 128), jnp.float32),
        pltpu.SemaphoreType.DMA((2,)),
        pltpu.SemaphoreType.DMA((2,)),
    ],
)
def _emb_lookup(
    ids_hbm, table_hbm, out_hbm, idx_v, gbuf0, gbuf1, tbuf0, tbuf1, gsem, osem
):
    gbufs = (gbuf0, gbuf1)
    tbufs = (tbuf0, tbuf1)
    wid = lax.axis_index("s") * NUM_CORES + lax.axis_index("c")
    pltpu.sync_copy(ids_hbm.at[:, pl.ds(wid * BBLK, BBLK)], idx_v)

    iota16 = lax.iota(jnp.int32, 16)

    def g_copy(t, s):
        return pltpu.make_async_copy(
            table_hbm.at[idx_v.at[t]], gbufs[s], gsem.at[s]
        )

    def o_copies(t, s):
        return [
            pltpu.make_async_copy(
                tbufs[s].at[hb], out_hbm.at[t, hb, wid], osem.at[s]
            )
            for hb in range(HB)
        ]

    hb_idx = [(16 * k + iota16) // 8 for k in range(HIDDEN // 16)]
    h8_idx = [(16 * k + iota16) % 8 for k in range(HIDDEN // 16)]

    def transpose(s):
        gbuf = gbufs[s]
        tbuf = tbufs[s]

        def j_body(j, _):
            j_v = jnp.full((16,), 0, jnp.int32) + j
            for k in range(HIDDEN // 16):
                v = gbuf[j, pl.ds(16 * k, 16)]
                plsc.store_scatter(tbuf, [hb_idx[k], h8_idx[k], j_v], v)
            return ()

        lax.fori_loop(0, BBLK, j_body, (), unroll=2)

    def step(t, s, *, first=False, last=False):
        g_copy(t, s).wait()
        if not last:
            g_copy(t + 1, 1 - s).start()
        if not first:
            for c in o_copies(t - 2, s):
                c.wait()
        transpose(s)
        for c in o_copies(t, s):
            c.start()

    g_copy(0, 0).start()
    step(0, 0, first=True)
    step(1, 1, first=True)

    def steady(t2, _):
        t = t2 * 2
        step(t, 0)
        step(t + 1, 1)
        return ()

    lax.fori_loop(1, (HIST - 2) // 2, steady, (), unroll=False)

    step(HIST - 2, 0)
    step(HIST - 1, 1, last=True)
    for s in (0, 1):
        for c in o_copies(HIST - 2 + s, s):
            c.wait()


def kernel(weight, input_ids):
    ids_t = input_ids.T.astype(jnp.int32)
    z = _emb_lookup(ids_t, weight)
    return z.transpose(2, 4, 0, 1, 3).reshape(BATCH, HIST, HIDDEN)


# speedup vs baseline: 1.8220x; 1.5975x over previous
import functools

import jax
import jax.numpy as jnp
from jax import lax
from jax.experimental import pallas as pl
from jax.experimental.pallas import tpu as pltpu
from jax.experimental.pallas import tpu_sc as plsc

VOCAB = 1_000_000
HIDDEN = 64
BATCH = 4096
HIST = 200

NUM_CORES = 2
NUM_SUBCORES = 16
NW = NUM_CORES * NUM_SUBCORES
BBLK = BATCH // NW
HB = HIDDEN // 8

_mesh = plsc.VectorSubcoreMesh(core_axis_name="c", subcore_axis_name="s")


@functools.partial(
    pl.kernel,
    mesh=_mesh,
    compiler_params=pltpu.CompilerParams(
        use_tc_tiling_on_sc=False, needs_layout_passes=False
    ),
    out_type=jax.ShapeDtypeStruct((HIST, HB, NW, 8, 128), jnp.float32),
    scratch_types=[
        pltpu.VMEM((HIST, BBLK), jnp.int32),
        pltpu.VMEM((BBLK, HIDDEN), jnp.float32),
        pltpu.VMEM((BBLK, HIDDEN), jnp.float32),
        pltpu.VMEM((HB, 8, 129), jnp.float32),
        pltpu.VMEM((HB, 8, 129), jnp.float32),
        pltpu.SemaphoreType.DMA((2,)),
        pltpu.SemaphoreType.DMA((2,)),
    ],
)
def _emb_lookup(
    ids_hbm, table_hbm, out_hbm, idx_v, gbuf0, gbuf1, tbuf0, tbuf1, gsem, osem
):
    gbufs = (gbuf0, gbuf1)
    tbufs = (tbuf0, tbuf1)
    wid = lax.axis_index("s") * NUM_CORES + lax.axis_index("c")
    pltpu.sync_copy(ids_hbm.at[:, pl.ds(wid * BBLK, BBLK)], idx_v)

    iota16 = lax.iota(jnp.int32, 16)

    def g_copy(t, s):
        return pltpu.make_async_copy(
            table_hbm.at[idx_v.at[t]], gbufs[s], gsem.at[s]
        )

    def o_copies(t, s):
        return [
            pltpu.make_async_copy(
                tbufs[s].at[hb, :, pl.ds(0, 128)], out_hbm.at[t, hb, wid], osem.at[s]
            )
            for hb in range(HB)
        ]

    hb_idx = [(16 * k + iota16) // 8 for k in range(HIDDEN // 16)]
    h8_idx = [(16 * k + iota16) % 8 for k in range(HIDDEN // 16)]

    def transpose(s):
        gbuf = gbufs[s]
        tbuf = tbufs[s]

        def j_body(j, _):
            j_v = jnp.full((16,), 0, jnp.int32) + j
            for k in range(HIDDEN // 16):
                v = gbuf[j, pl.ds(16 * k, 16)]
                plsc.store_scatter(tbuf, [hb_idx[k], h8_idx[k], j_v], v)
            return ()

        lax.fori_loop(0, BBLK, j_body, (), unroll=2)

    def step(t, s, *, first=False, last=False):
        g_copy(t, s).wait()
        if not last:
            g_copy(t + 1, 1 - s).start()
        if not first:
            for c in o_copies(t - 2, s):
                c.wait()
        transpose(s)
        for c in o_copies(t, s):
            c.start()

    g_copy(0, 0).start()
    step(0, 0, first=True)
    step(1, 1, first=True)

    def steady(t2, _):
        t = t2 * 2
        step(t, 0)
        step(t + 1, 1)
        return ()

    lax.fori_loop(1, (HIST - 2) // 2, steady, (), unroll=False)

    step(HIST - 2, 0)
    step(HIST - 1, 1, last=True)
    for s in (0, 1):
        for c in o_copies(HIST - 2 + s, s):
            c.wait()


def kernel(weight, input_ids):
    ids_t = input_ids.T.astype(jnp.int32)
    z = _emb_lookup(ids_t, weight)
    return z.transpose(2, 4, 0, 1, 3).reshape(BATCH, HIST, HIDDEN)


# speedup vs baseline: 1.8463x; 1.0133x over previous
import functools

import jax
import jax.numpy as jnp
from jax import lax
from jax.experimental import pallas as pl
from jax.experimental.pallas import tpu as pltpu
from jax.experimental.pallas import tpu_sc as plsc

VOCAB = 1_000_000
HIDDEN = 64
BATCH = 4096
HIST = 200

NUM_CORES = 2
NUM_SUBCORES = 16
NW = NUM_CORES * NUM_SUBCORES
BBLK = BATCH // NW
HB = HIDDEN // 8

_mesh = plsc.VectorSubcoreMesh(core_axis_name="c", subcore_axis_name="s")


@functools.partial(
    pl.kernel,
    mesh=_mesh,
    compiler_params=pltpu.CompilerParams(
        use_tc_tiling_on_sc=False, needs_layout_passes=False
    ),
    out_type=jax.ShapeDtypeStruct((HIST, HB, NW, 8, 128), jnp.float32),
    scratch_types=[
        pltpu.VMEM((HIST, BBLK), jnp.int32),
        pltpu.VMEM((BBLK, HIDDEN), jnp.float32),
        pltpu.VMEM((BBLK, HIDDEN), jnp.float32),
        pltpu.VMEM((HB, 8, 129), jnp.float32),
        pltpu.VMEM((HB, 8, 129), jnp.float32),
        pltpu.SemaphoreType.DMA((2,)),
        pltpu.SemaphoreType.DMA((2,)),
    ],
)
def _emb_lookup(
    ids_hbm, table_hbm, out_hbm, idx_v, gbuf0, gbuf1, tbuf0, tbuf1, gsem, osem
):
    gbufs = (gbuf0, gbuf1)
    tbufs = (tbuf0, tbuf1)
    wid = lax.axis_index("s") * NUM_CORES + lax.axis_index("c")
    pltpu.sync_copy(ids_hbm.at[:, pl.ds(wid * BBLK, BBLK)], idx_v)

    iota16 = lax.iota(jnp.int32, 16)

    def g_copy(t, s):
        return pltpu.make_async_copy(
            table_hbm.at[idx_v.at[t]], gbufs[s], gsem.at[s]
        )

    def o_copies(t, s):
        return [
            pltpu.make_async_copy(
                tbufs[s].at[hb, :, pl.ds(0, 128)], out_hbm.at[t, hb, wid], osem.at[s]
            )
            for hb in range(HB)
        ]

    hb_idx = [(16 * k + iota16) // 8 for k in range(HIDDEN // 16)]
    h8_idx = [(16 * k + iota16) % 8 for k in range(HIDDEN // 16)]

    def transpose(s):
        gbuf = gbufs[s]
        tbuf = tbufs[s]

        def j_body(j, _):
            j_v = jnp.full((16,), 0, jnp.int32) + j
            for k in range(HIDDEN // 16):
                v = gbuf[j, pl.ds(16 * k, 16)]
                plsc.store_scatter(tbuf, [hb_idx[k], h8_idx[k], j_v], v)
            return ()

        lax.fori_loop(0, BBLK, j_body, (), unroll=8)

    def step(t, s, *, first=False, last=False):
        g_copy(t, s).wait()
        if not last:
            g_copy(t + 1, 1 - s).start()
        if not first:
            for c in o_copies(t - 2, s):
                c.wait()
        transpose(s)
        for c in o_copies(t, s):
            c.start()

    g_copy(0, 0).start()
    step(0, 0, first=True)
    step(1, 1, first=True)

    def steady(t2, _):
        t = t2 * 2
        step(t, 0)
        step(t + 1, 1)
        return ()

    lax.fori_loop(1, (HIST - 2) // 2, steady, (), unroll=False)

    step(HIST - 2, 0)
    step(HIST - 1, 1, last=True)
    for s in (0, 1):
        for c in o_copies(HIST - 2 + s, s):
            c.wait()


def kernel(weight, input_ids):
    ids_t = input_ids.T.astype(jnp.int32)
    z = _emb_lookup(ids_t, weight)
    return z.transpose(2, 4, 0, 1, 3).reshape(BATCH, HIST, HIDDEN)
